# initial kernel scaffold (unmeasured)
import jax
import jax.numpy as jnp
from jax import lax
from jax.experimental import pallas as pl
from jax.experimental.pallas import tpu as pltpu

N_DEV = 4
M = 2048
N = 2048
CHUNK = M // N_DEV


def kernel(x, w_mat):
    def body(x_ref, w_ref, out_ref, send_buf, recv_buf, send_sems, recv_sems):
        my = lax.axis_index("i")
        left = lax.rem(my + N_DEV - 1, N_DEV)
        right = lax.rem(my + 1, N_DEV)

        barrier = pltpu.get_barrier_semaphore()
        for nbr in (left, right):
            pl.semaphore_signal(
                barrier, inc=1, device_id=(nbr,),
                device_id_type=pl.DeviceIdType.MESH,
            )
        pl.semaphore_wait(barrier, 2)

        out_ref[:, :] = jnp.dot(
            x_ref[:, :], w_ref[:, :], preferred_element_type=jnp.float32
        )

        def hop(h, s_idx, r_idx):
            slot = h % 2
            send_buf[slot, :, :] = out_ref[pl.ds(s_idx * CHUNK, CHUNK), :]
            rdma = pltpu.make_async_remote_copy(
                src_ref=send_buf.at[slot],
                dst_ref=recv_buf.at[slot],
                send_sem=send_sems.at[slot],
                recv_sem=recv_sems.at[slot],
                device_id=(right,),
                device_id_type=pl.DeviceIdType.MESH,
            )
            rdma.start()
            rdma.wait()
            return r_idx

        for h in range(N_DEV - 1):
            s_idx = lax.rem(my - h + 2 * N_DEV, N_DEV)
            r_idx = lax.rem(my - h - 1 + 2 * N_DEV, N_DEV)
            hop(h, s_idx, r_idx)
            out_ref[pl.ds(r_idx * CHUNK, CHUNK), :] += recv_buf[h % 2, :, :]

        own = lax.rem(my + 1, N_DEV)
        out_ref[pl.ds(own * CHUNK, CHUNK), :] = jnp.maximum(
            out_ref[pl.ds(own * CHUNK, CHUNK), :], 0.0
        )

        for g in range(N_DEV - 1):
            h = (N_DEV - 1) + g
            s_idx = lax.rem(my + 1 - g + 2 * N_DEV, N_DEV)
            r_idx = lax.rem(my - g + 2 * N_DEV, N_DEV)
            hop(h, s_idx, r_idx)
            out_ref[pl.ds(r_idx * CHUNK, CHUNK), :] = recv_buf[h % 2, :, :]

    return pl.pallas_call(
        body,
        out_shape=jax.ShapeDtypeStruct((M, N), jnp.float32),
        in_specs=[
            pl.BlockSpec(memory_space=pltpu.VMEM),
            pl.BlockSpec(memory_space=pltpu.VMEM),
        ],
        out_specs=pl.BlockSpec(memory_space=pltpu.VMEM),
        scratch_shapes=[
            pltpu.VMEM((2, CHUNK, N), jnp.float32),
            pltpu.VMEM((2, CHUNK, N), jnp.float32),
            pltpu.SemaphoreType.DMA((2,)),
            pltpu.SemaphoreType.DMA((2,)),
        ],
        compiler_params=pltpu.CompilerParams(collective_id=0),
    )(x, w_mat)


# baseline (device time: 313574 ns/iter reference)
import jax
import jax.numpy as jnp
from jax import lax
from jax.experimental import pallas as pl
from jax.experimental.pallas import tpu as pltpu

N_DEV = 4
M = 2048
N = 2048
CHUNK = M // N_DEV


def kernel(x, w_mat):
    def body(x_ref, w_ref, out_ref, send_buf, recv_buf, send_sems, recv_sems):
        my = lax.axis_index("i")
        left = lax.rem(my + N_DEV - 1, N_DEV)
        right = lax.rem(my + 1, N_DEV)

        barrier = pltpu.get_barrier_semaphore()
        for nbr in (left, right):
            pl.semaphore_signal(
                barrier, inc=1, device_id=(nbr,),
                device_id_type=pl.DeviceIdType.MESH,
            )
        pl.semaphore_wait(barrier, 2)

        out_ref[:, :] = jnp.dot(
            x_ref[:, :], w_ref[:, :], preferred_element_type=jnp.float32
        )

        def hop(h, s_idx, r_idx):
            slot = h % 2
            send_buf[slot, :, :] = out_ref[pl.ds(s_idx * CHUNK, CHUNK), :]
            rdma = pltpu.make_async_remote_copy(
                src_ref=send_buf.at[slot],
                dst_ref=recv_buf.at[slot],
                send_sem=send_sems.at[slot],
                recv_sem=recv_sems.at[slot],
                device_id=(right,),
                device_id_type=pl.DeviceIdType.MESH,
            )
            rdma.start()
            rdma.wait()
            return r_idx

        for h in range(N_DEV - 1):
            s_idx = lax.rem(my - h + 2 * N_DEV, N_DEV)
            r_idx = lax.rem(my - h - 1 + 2 * N_DEV, N_DEV)
            hop(h, s_idx, r_idx)
            out_ref[pl.ds(r_idx * CHUNK, CHUNK), :] += recv_buf[h % 2, :, :]

        own = lax.rem(my + 1, N_DEV)
        out_ref[pl.ds(own * CHUNK, CHUNK), :] = jnp.maximum(
            out_ref[pl.ds(own * CHUNK, CHUNK), :], 0.0
        )

        for g in range(N_DEV - 1):
            h = (N_DEV - 1) + g
            s_idx = lax.rem(my + 1 - g + 2 * N_DEV, N_DEV)
            r_idx = lax.rem(my - g + 2 * N_DEV, N_DEV)
            hop(h, s_idx, r_idx)
            out_ref[pl.ds(r_idx * CHUNK, CHUNK), :] = recv_buf[h % 2, :, :]

    return pl.pallas_call(
        body,
        out_shape=jax.ShapeDtypeStruct((M, N), jnp.float32),
        in_specs=[
            pl.BlockSpec(memory_space=pltpu.VMEM),
            pl.BlockSpec(memory_space=pltpu.VMEM),
        ],
        out_specs=pl.BlockSpec(memory_space=pltpu.VMEM),
        scratch_shapes=[
            pltpu.VMEM((2, CHUNK, N), jnp.float32),
            pltpu.VMEM((2, CHUNK, N), jnp.float32),
            pltpu.SemaphoreType.DMA((2,)),
            pltpu.SemaphoreType.DMA((2,)),
        ],
        compiler_params=pltpu.CompilerParams(
            collective_id=0, vmem_limit_bytes=100 * 1024 * 1024
        ),
    )(x, w_mat)


# device time: 172470 ns/iter; 1.8181x vs baseline; 1.8181x over previous
import jax
import jax.numpy as jnp
from jax import lax
from jax.experimental import pallas as pl
from jax.experimental.pallas import tpu as pltpu

N_DEV = 4
M = 2048
N = 2048
CHUNK = M // N_DEV
HALF = N // 2


def kernel(x, w_mat):
    def body(x_ref, w_ref, out_ref, sbuf, rbuf, ssems, rsems):
        my = lax.axis_index("i")
        left = lax.rem(my + N_DEV - 1, N_DEV)
        right = lax.rem(my + 1, N_DEV)

        def m4(v):
            return lax.rem(v + 2 * N_DEV, N_DEV)

        barrier = pltpu.get_barrier_semaphore()
        for nbr in (left, right):
            pl.semaphore_signal(
                barrier, inc=1, device_id=(nbr,),
                device_id_type=pl.DeviceIdType.MESH,
            )
        pl.semaphore_wait(barrier, 2)

        def rows(idx):
            return pl.ds(idx * CHUNK, CHUNK)

        CW = pl.ds(0, HALF)
        CCW = pl.ds(HALF, HALF)

        def compute_chunk(idx):
            out_ref[rows(idx), :] = jnp.dot(
                x_ref[rows(idx), :], w_ref[:, :],
                preferred_element_type=jnp.float32,
            )

        def make_rdma(dirn, slot, src, tgt):
            return pltpu.make_async_remote_copy(
                src_ref=src,
                dst_ref=rbuf.at[dirn, slot],
                send_sem=ssems.at[dirn, slot],
                recv_sem=rsems.at[dirn, slot],
                device_id=(tgt,),
                device_id_type=pl.DeviceIdType.MESH,
            )

        def start_pair(slot, src0, src1):
            r0 = make_rdma(0, slot, src0, right)
            r1 = make_rdma(1, slot, src1, left)
            r0.start()
            r1.start()
            return r0, r1

        compute_chunk(my)
        sbuf[0, 0, :, :] = out_ref[rows(my), CW]
        sbuf[1, 0, :, :] = out_ref[rows(my), CCW]
        r0, r1 = start_pair(0, sbuf.at[0, 0], sbuf.at[1, 0])
        for j in range(1, N_DEV):
            compute_chunk(m4(my + j))
        r0.wait()
        r1.wait()

        for h in (1, 2):
            slot, prev = h % 2, (h - 1) % 2
            a_cw = m4(my - h)
            a_ccw = m4(my + h)
            sbuf[0, slot, :, :] = out_ref[rows(a_cw), CW] + rbuf[0, prev, :, :]
            sbuf[1, slot, :, :] = out_ref[rows(a_ccw), CCW] + rbuf[1, prev, :, :]
            r0, r1 = start_pair(slot, sbuf.at[0, slot], sbuf.at[1, slot])
            out_ref[rows(a_cw), CW] = sbuf[0, slot]
            out_ref[rows(a_ccw), CCW] = sbuf[1, slot]
            r0.wait()
            r1.wait()

        own_cw = m4(my + 1)
        own_ccw = m4(my - 1)
        sbuf[0, 1, :, :] = jnp.maximum(out_ref[rows(own_cw), CW] + rbuf[0, 0, :, :], 0.0)
        sbuf[1, 1, :, :] = jnp.maximum(out_ref[rows(own_ccw), CCW] + rbuf[1, 0, :, :], 0.0)
        r0, r1 = start_pair(1, sbuf.at[0, 1], sbuf.at[1, 1])
        out_ref[rows(own_cw), CW] = sbuf[0, 1]
        out_ref[rows(own_ccw), CCW] = sbuf[1, 1]
        r0.wait()
        r1.wait()

        for g in (1, 2):
            slot, prev = (3 + g) % 2, (2 + g) % 2
            r0, r1 = start_pair(slot, rbuf.at[0, prev], rbuf.at[1, prev])
            out_ref[rows(m4(my - g + 1)), CW] = rbuf[0, prev, :, :]
            out_ref[rows(m4(my + g - 1)), CCW] = rbuf[1, prev, :, :]
            r0.wait()
            r1.wait()
        out_ref[rows(m4(my - 2)), CW] = rbuf[0, 1, :, :]
        out_ref[rows(m4(my + 2)), CCW] = rbuf[1, 1, :, :]

    return pl.pallas_call(
        body,
        out_shape=jax.ShapeDtypeStruct((M, N), jnp.float32),
        in_specs=[
            pl.BlockSpec(memory_space=pltpu.VMEM),
            pl.BlockSpec(memory_space=pltpu.VMEM),
        ],
        out_specs=pl.BlockSpec(memory_space=pltpu.VMEM),
        scratch_shapes=[
            pltpu.VMEM((2, 2, CHUNK, HALF), jnp.float32),
            pltpu.VMEM((2, 2, CHUNK, HALF), jnp.float32),
            pltpu.SemaphoreType.DMA((2, 2)),
            pltpu.SemaphoreType.DMA((2, 2)),
        ],
        compiler_params=pltpu.CompilerParams(
            collective_id=0, vmem_limit_bytes=100 * 1024 * 1024
        ),
    )(x, w_mat)


# device time: 162842 ns/iter; 1.9256x vs baseline; 1.0591x over previous
import jax
import jax.numpy as jnp
from jax import lax
from jax.experimental import pallas as pl
from jax.experimental.pallas import tpu as pltpu

N_DEV = 4
M = 2048
N = 2048
CHUNK = M // N_DEV
QCOL = N // 4
STREAMS = (0, 2, 1, 3)


def kernel(x, w_mat):
    def body(x_ref, w_ref, out_ref, sbuf, rbuf, ssems, rsems):
        my = lax.axis_index("i")
        left = lax.rem(my + N_DEV - 1, N_DEV)
        right = lax.rem(my + 1, N_DEV)

        def m4(v):
            return lax.rem(v + 2 * N_DEV, N_DEV)

        barrier = pltpu.get_barrier_semaphore()
        for nbr in (left, right):
            pl.semaphore_signal(
                barrier, inc=1, device_id=(nbr,),
                device_id_type=pl.DeviceIdType.MESH,
            )
        pl.semaphore_wait(barrier, 2)

        def rows(idx):
            return pl.ds(idx * CHUNK, CHUNK)

        def cols(s):
            return pl.ds(s * QCOL, QCOL)

        def tgt(s):
            return right if s < 2 else left

        def sgn(s):
            return 1 if s < 2 else -1

        def make_rdma(s, slot, src):
            return pltpu.make_async_remote_copy(
                src_ref=src,
                dst_ref=rbuf.at[s, slot],
                send_sem=ssems.at[s, slot],
                recv_sem=rsems.at[s, slot],
                device_id=(tgt(s),),
                device_id_type=pl.DeviceIdType.MESH,
            )

        hop0 = {}
        for s in STREAMS:
            sbuf[s, 0, :, :] = jnp.dot(
                x_ref[rows(my), :], w_ref[:, cols(s)],
                preferred_element_type=jnp.float32,
            )
            r = make_rdma(s, 0, sbuf.at[s, 0])
            r.start()
            hop0[s] = r
        for s in STREAMS:
            out_ref[rows(my), cols(s)] = sbuf[s, 0, :, :]
        for j in range(1, N_DEV):
            idx = m4(my + j)
            out_ref[rows(idx), :] = jnp.dot(
                x_ref[rows(idx), :], w_ref[:, :],
                preferred_element_type=jnp.float32,
            )

        prev_rdma = hop0
        for h in (1, 2):
            slot, prev = h % 2, (h - 1) % 2
            cur = {}
            for s in STREAMS:
                a = m4(my - sgn(s) * h)
                prev_rdma[s].wait()
                sbuf[s, slot, :, :] = (
                    out_ref[rows(a), cols(s)] + rbuf[s, prev, :, :]
                )
                r = make_rdma(s, slot, sbuf.at[s, slot])
                r.start()
                cur[s] = r
                out_ref[rows(a), cols(s)] = sbuf[s, slot, :, :]
            prev_rdma = cur

        cur = {}
        for s in STREAMS:
            own = m4(my + sgn(s))
            prev_rdma[s].wait()
            sbuf[s, 1, :, :] = jnp.maximum(
                out_ref[rows(own), cols(s)] + rbuf[s, 0, :, :], 0.0
            )
            r = make_rdma(s, 1, sbuf.at[s, 1])
            r.start()
            cur[s] = r
            out_ref[rows(own), cols(s)] = sbuf[s, 1, :, :]
        prev_rdma = cur

        for g in (1, 2):
            slot, prev = (3 + g) % 2, (2 + g) % 2
            cur = {}
            for s in STREAMS:
                prev_rdma[s].wait()
                r = make_rdma(s, slot, rbuf.at[s, prev])
                r.start()
                cur[s] = r
                out_ref[rows(m4(my - sgn(s) * (g - 1))), cols(s)] = (
                    rbuf[s, prev, :, :]
                )
            prev_rdma = cur
        for s in STREAMS:
            prev_rdma[s].wait()
            out_ref[rows(m4(my - sgn(s) * 2)), cols(s)] = rbuf[s, 1, :, :]

    return pl.pallas_call(
        body,
        out_shape=jax.ShapeDtypeStruct((M, N), jnp.float32),
        in_specs=[
            pl.BlockSpec(memory_space=pltpu.VMEM),
            pl.BlockSpec(memory_space=pltpu.VMEM),
        ],
        out_specs=pl.BlockSpec(memory_space=pltpu.VMEM),
        scratch_shapes=[
            pltpu.VMEM((4, 2, CHUNK, QCOL), jnp.float32),
            pltpu.VMEM((4, 2, CHUNK, QCOL), jnp.float32),
            pltpu.SemaphoreType.DMA((4, 2)),
            pltpu.SemaphoreType.DMA((4, 2)),
        ],
        compiler_params=pltpu.CompilerParams(
            collective_id=0, vmem_limit_bytes=100 * 1024 * 1024
        ),
    )(x, w_mat)
